# Initial kernel scaffold; baseline (speedup 1.0000x reference)
import jax
import jax.numpy as jnp
from jax.experimental import pallas as pl


def kernel(x, edge_index, W1, b1, W2, b2):
    raise NotImplementedError("write your pallas kernel here")


# speedup vs baseline: 5.5698x; 5.5698x over previous
import functools

import jax
import jax.numpy as jnp
from jax import lax
from jax.experimental import pallas as pl
from jax.experimental.pallas import tpu as pltpu
from jax.experimental.pallas import tpu_sc as plsc

N = 10000
E = 320000
D = 128
K = 10
ALPHA = 0.1

NC = 2
NS = 16
HALF = N // NC
RPT = 320
AGG_ROWS = NS * RPT
GARBAGE = HALF
U = 128
E_PAD = E + 2 * U
EPU = E_PAD // U
RCH = 64
ROW_OFFS = (0, 64, 128, 192, 256)

_mesh = plsc.VectorSubcoreMesh(core_axis_name="c", subcore_axis_name="s")


def _row_base(s):
    return jnp.minimum(s * RPT, HALF - RPT)


@functools.partial(
    pl.kernel,
    out_type=jax.ShapeDtypeStruct((N, 16), jnp.float32),
    mesh=_mesh,
    scratch_types=[
        pltpu.VMEM((32, 16), jnp.int32),
        pltpu.VMEM((U, 16), jnp.float32),
        pltpu.VMEM((1, U), jnp.int32),
        pltpu.VMEM((RCH, 16), jnp.float32),
        pltpu.VMEM_SHARED((AGG_ROWS, 16), jnp.float32),
    ],
)
def _deg_kernel(cols_hbm, meta_hbm, deg_hbm, meta_v, ones_v, cidx_v, chunk_v,
                deg_sh):
    c = lax.axis_index("c")
    s = lax.axis_index("s")
    w = c * NS + s
    pltpu.sync_copy(meta_hbm, meta_v)

    def fill(r, _):
        ones_v[r, :] = jnp.full((16,), 1.0, jnp.float32)
        return 0

    lax.fori_loop(0, U, fill, 0)

    def zfill(r, _):
        chunk_v[r, :] = jnp.zeros((16,), jnp.float32)
        return 0

    lax.fori_loop(0, RCH, zfill, 0)

    for off in ROW_OFFS:
        pltpu.sync_copy(chunk_v, deg_sh.at[pl.ds(s * RPT + off, RCH)])
    plsc.subcore_barrier()

    mrow = meta_v[w, :]
    u0 = mrow[0]
    nu = mrow[1]

    def edge_unit(i, _):
        pltpu.sync_copy(cols_hbm.at[pl.ds(u0 + i, 1)], cidx_v)
        pltpu.sync_copy(ones_v, deg_sh.at[cidx_v.at[0]], add=True)
        return 0

    lax.fori_loop(0, nu, edge_unit, 0)
    plsc.subcore_barrier()

    lr = _row_base(s)
    for off in ROW_OFFS:
        pltpu.sync_copy(deg_sh.at[pl.ds(lr + off, RCH)], chunk_v)
        pltpu.sync_copy(chunk_v, deg_hbm.at[pl.ds(c * HALF + lr + off, RCH)])


@functools.partial(
    pl.kernel,
    out_type=jax.ShapeDtypeStruct((N, D), jnp.float32),
    mesh=_mesh,
    scratch_types=[
        pltpu.VMEM((32, 16), jnp.int32),
        pltpu.VMEM((1, U), jnp.int32),
        pltpu.VMEM((1, U), jnp.int32),
        pltpu.VMEM((U, D), jnp.float32),
        pltpu.VMEM((RCH, D), jnp.float32),
        pltpu.VMEM((RCH, D), jnp.float32),
        pltpu.VMEM((RCH, D), jnp.float32),
        pltpu.VMEM((RCH, D), jnp.float32),
        pltpu.VMEM((RCH, 16), jnp.float32),
        pltpu.VMEM((RCH, D), jnp.float32),
        pltpu.VMEM_SHARED((AGG_ROWS, D), jnp.float32),
        pltpu.SemaphoreType.DMA,
    ],
)
def _prop_kernel(g_hbm, rows_hbm, cols_hbm, a_hbm, c_hbm, meta_hbm, gout_hbm,
                 meta_v, ridx_v, cidx_v, gbuf_v, zero_v, agg_v, gold_v,
                 cvec_v, avec_v, gnew_v, agg_sh, sem):
    c = lax.axis_index("c")
    s = lax.axis_index("s")
    w = c * NS + s
    pltpu.sync_copy(meta_hbm, meta_v)

    def zfill(r, _):
        for k in range(D // 16):
            zero_v[r, pl.ds(k * 16, 16)] = jnp.zeros((16,), jnp.float32)
        return 0

    lax.fori_loop(0, RCH, zfill, 0)
    for off in ROW_OFFS:
        pltpu.sync_copy(zero_v, agg_sh.at[pl.ds(s * RPT + off, RCH)])
    plsc.subcore_barrier()

    mrow = meta_v[w, :]
    u0 = mrow[0]
    nu = mrow[1]

    def edge_unit(i, _):
        pltpu.sync_copy(rows_hbm.at[pl.ds(u0 + i, 1)], ridx_v)
        pltpu.sync_copy(cols_hbm.at[pl.ds(u0 + i, 1)], cidx_v)
        pltpu.async_copy(g_hbm.at[ridx_v.at[0]], gbuf_v, sem).wait()
        pltpu.sync_copy(gbuf_v, agg_sh.at[cidx_v.at[0]], add=True)
        return 0

    lax.fori_loop(0, nu, edge_unit, 0)
    plsc.subcore_barrier()

    lr = _row_base(s)
    for off in ROW_OFFS:
        gr = c * HALF + lr + off
        pltpu.sync_copy(agg_sh.at[pl.ds(lr + off, RCH)], agg_v)
        pltpu.sync_copy(g_hbm.at[pl.ds(gr, RCH)], gold_v)
        pltpu.sync_copy(c_hbm.at[pl.ds(gr, RCH)], cvec_v)
        pltpu.sync_copy(a_hbm.at[pl.ds(gr, RCH)], avec_v)

        def rowfn(r, _):
            a_s = avec_v[r, :][0]
            for k in range(D // 16):
                sl = pl.ds(k * 16, 16)
                gnew_v[r, sl] = a_s * (agg_v[r, sl] + gold_v[r, sl]) \
                    + cvec_v[r, sl]
            return 0

        lax.fori_loop(0, RCH, rowfn, 0)
        pltpu.sync_copy(gnew_v, gout_hbm.at[pl.ds(gr, RCH)])


BLK = 1000


def _mlp_body(x_ref, deg_ref, w1_ref, b1_ref, w2_ref, b2_ref,
              g0_ref, a_ref, c_ref, dinv_ref):
    x = x_ref[...]
    h = jnp.dot(x, w1_ref[...].T, preferred_element_type=jnp.float32)
    h = jnp.maximum(h + b1_ref[...], 0.0)
    h = jnp.dot(h, w2_ref[...].T, preferred_element_type=jnp.float32)
    h = h + b2_ref[...]
    deg = deg_ref[...][:, 0:1] + 1.0
    dinv = lax.rsqrt(deg)
    g0 = h * dinv
    g0_ref[...] = g0
    a_ref[...] = jnp.broadcast_to((1.0 - ALPHA) * dinv * dinv, (BLK, 16))
    c_ref[...] = ALPHA * g0
    dinv_ref[...] = dinv


def _mlp_stage(x, deg16, W1, b1, W2, b2):
    grid = (N // BLK,)
    return pl.pallas_call(
        _mlp_body,
        grid=grid,
        in_specs=[
            pl.BlockSpec((BLK, D), lambda i: (i, 0)),
            pl.BlockSpec((BLK, 16), lambda i: (i, 0)),
            pl.BlockSpec((D, D), lambda i: (0, 0)),
            pl.BlockSpec((1, D), lambda i: (0, 0)),
            pl.BlockSpec((D, D), lambda i: (0, 0)),
            pl.BlockSpec((1, D), lambda i: (0, 0)),
        ],
        out_specs=[
            pl.BlockSpec((BLK, D), lambda i: (i, 0)),
            pl.BlockSpec((BLK, 16), lambda i: (i, 0)),
            pl.BlockSpec((BLK, D), lambda i: (i, 0)),
            pl.BlockSpec((BLK, 1), lambda i: (i, 0)),
        ],
        out_shape=[
            jax.ShapeDtypeStruct((N, D), jnp.float32),
            jax.ShapeDtypeStruct((N, 16), jnp.float32),
            jax.ShapeDtypeStruct((N, D), jnp.float32),
            jax.ShapeDtypeStruct((N, 1), jnp.float32),
        ],
    )(x, deg16, W1, b1.reshape(1, D), W2, b2.reshape(1, D))


def _out_body(g_ref, dinv_ref, o_ref):
    h = g_ref[...] / dinv_ref[...]
    m = jnp.max(h, axis=1, keepdims=True)
    ex = jnp.exp(h - m)
    lse = jnp.log(jnp.sum(ex, axis=1, keepdims=True))
    o_ref[...] = h - m - lse


def _out_stage(g, dinv):
    grid = (N // BLK,)
    return pl.pallas_call(
        _out_body,
        grid=grid,
        in_specs=[
            pl.BlockSpec((BLK, D), lambda i: (i, 0)),
            pl.BlockSpec((BLK, 1), lambda i: (i, 0)),
        ],
        out_specs=pl.BlockSpec((BLK, D), lambda i: (i, 0)),
        out_shape=jax.ShapeDtypeStruct((N, D), jnp.float32),
    )(g, dinv)


def _prep_edges(edge_index):
    row = edge_index[0].astype(jnp.int32)
    col = edge_index[1].astype(jnp.int32)
    in0 = col < HALF
    n0 = jnp.sum(in0.astype(jnp.int32))
    pad0 = (-n0) % U
    b0p = n0 + pad0
    pos0 = jnp.cumsum(in0.astype(jnp.int32)) - 1
    pos1 = b0p + jnp.cumsum((~in0).astype(jnp.int32)) - 1
    pos = jnp.where(in0, pos0, pos1)
    rows_p = jnp.zeros((E_PAD,), jnp.int32).at[pos].set(row)
    col_local = col - jnp.where(in0, 0, HALF).astype(jnp.int32)
    cols_p = jnp.full((E_PAD,), GARBAGE, jnp.int32).at[pos].set(col_local)

    t0 = b0p // U
    t1 = EPU - t0
    sar = jnp.arange(NS + 1, dtype=jnp.int32)
    bnd0 = (sar * t0) // NS
    bnd1 = t0 + (sar * t1) // NS
    starts = jnp.concatenate([bnd0[:-1], bnd1[:-1]])
    nums = jnp.concatenate([bnd0[1:] - bnd0[:-1], bnd1[1:] - bnd1[:-1]])
    meta = jnp.zeros((32, 16), jnp.int32)
    meta = meta.at[:, 0].set(starts).at[:, 1].set(nums)
    return (rows_p.reshape(EPU, U), cols_p.reshape(EPU, U), meta)


def kernel(x, edge_index, W1, b1, W2, b2):
    rows2d, cols2d, meta = _prep_edges(edge_index)
    deg16 = _deg_kernel(cols2d, meta)
    g, avec, cvec, dinv = _mlp_stage(x, deg16, W1, b1, W2, b2)
    for _ in range(K):
        g = _prop_kernel(g, rows2d, cols2d, avec, cvec, meta)
    return _out_stage(g, dinv)


# speedup vs baseline: 5.8053x; 1.0423x over previous
import functools

import jax
import jax.numpy as jnp
from jax import lax
from jax.experimental import pallas as pl
from jax.experimental.pallas import tpu as pltpu
from jax.experimental.pallas import tpu_sc as plsc

N = 10000
E = 320000
D = 128
K = 10
ALPHA = 0.1

NC = 2
NS = 16
HALF = N // NC
RPT = 320
AGG_ROWS = NS * RPT
GARBAGE = HALF
U = 128
E_PAD = E + 2 * U
EPU = E_PAD // U
RCH = 64
ROW_OFFS = (0, 64, 128, 192, 256)

_mesh = plsc.VectorSubcoreMesh(core_axis_name="c", subcore_axis_name="s")


def _row_base(s):
    return jnp.minimum(s * RPT, HALF - RPT)


@functools.partial(
    pl.kernel,
    out_type=jax.ShapeDtypeStruct((N, 16), jnp.float32),
    mesh=_mesh,
    scratch_types=[
        pltpu.VMEM((32, 16), jnp.int32),
        pltpu.VMEM((U, 16), jnp.float32),
        pltpu.VMEM((1, U), jnp.int32),
        pltpu.VMEM((RCH, 16), jnp.float32),
        pltpu.VMEM_SHARED((AGG_ROWS, 16), jnp.float32),
    ],
)
def _deg_kernel(cols_hbm, meta_hbm, deg_hbm, meta_v, ones_v, cidx_v, chunk_v,
                deg_sh):
    c = lax.axis_index("c")
    s = lax.axis_index("s")
    w = c * NS + s
    pltpu.sync_copy(meta_hbm, meta_v)

    def fill(r, _):
        ones_v[r, :] = jnp.full((16,), 1.0, jnp.float32)
        return 0

    lax.fori_loop(0, U, fill, 0)

    def zfill(r, _):
        chunk_v[r, :] = jnp.zeros((16,), jnp.float32)
        return 0

    lax.fori_loop(0, RCH, zfill, 0)

    for off in ROW_OFFS:
        pltpu.sync_copy(chunk_v, deg_sh.at[pl.ds(s * RPT + off, RCH)])
    plsc.subcore_barrier()

    mrow = meta_v[w, :]
    u0 = mrow[0]
    nu = mrow[1]

    def edge_unit(i, _):
        pltpu.sync_copy(cols_hbm.at[pl.ds(u0 + i, 1)], cidx_v)
        pltpu.sync_copy(ones_v, deg_sh.at[cidx_v.at[0]], add=True)
        return 0

    lax.fori_loop(0, nu, edge_unit, 0)
    plsc.subcore_barrier()

    lr = _row_base(s)
    for off in ROW_OFFS:
        pltpu.sync_copy(deg_sh.at[pl.ds(lr + off, RCH)], chunk_v)
        pltpu.sync_copy(chunk_v, deg_hbm.at[pl.ds(c * HALF + lr + off, RCH)])


@functools.partial(
    pl.kernel,
    out_type=jax.ShapeDtypeStruct((N, D), jnp.float32),
    mesh=_mesh,
    scratch_types=[
        pltpu.VMEM((32, 16), jnp.int32),
        pltpu.VMEM((1, U), jnp.int32),
        pltpu.VMEM((1, U), jnp.int32),
        pltpu.VMEM((1, U), jnp.int32),
        pltpu.VMEM((1, U), jnp.int32),
        pltpu.VMEM((U, D), jnp.float32),
        pltpu.VMEM((U, D), jnp.float32),
        pltpu.VMEM((RCH, D), jnp.float32),
        pltpu.VMEM((RCH, D), jnp.float32),
        pltpu.VMEM((RCH, D), jnp.float32),
        pltpu.VMEM((RCH, D), jnp.float32),
        pltpu.VMEM((RCH, 16), jnp.float32),
        pltpu.VMEM((RCH, D), jnp.float32),
        pltpu.VMEM_SHARED((AGG_ROWS, D), jnp.float32),
        pltpu.SemaphoreType.DMA,
        pltpu.SemaphoreType.DMA,
        pltpu.SemaphoreType.DMA,
        pltpu.SemaphoreType.DMA,
    ],
)
def _prop_kernel(g_hbm, rows_hbm, cols_hbm, a_hbm, c_hbm, meta_hbm, gout_hbm,
                 meta_v, ridxa_v, ridxb_v, cidxa_v, cidxb_v, gbufa_v, gbufb_v,
                 zero_v, agg_v, gold_v, cvec_v, avec_v, gnew_v, agg_sh,
                 sga, sgb, ssa, ssb):
    c = lax.axis_index("c")
    s = lax.axis_index("s")
    w = c * NS + s
    pltpu.sync_copy(meta_hbm, meta_v)

    def zfill(r, _):
        for k in range(D // 16):
            zero_v[r, pl.ds(k * 16, 16)] = jnp.zeros((16,), jnp.float32)
        return 0

    lax.fori_loop(0, RCH, zfill, 0)
    for off in ROW_OFFS:
        pltpu.sync_copy(zero_v, agg_sh.at[pl.ds(s * RPT + off, RCH)])
    plsc.subcore_barrier()

    mrow = meta_v[w, :]
    u0 = mrow[0]
    nu = mrow[1]

    def edge_pair(j, _):
        u = u0 + 2 * j
        pltpu.sync_copy(rows_hbm.at[pl.ds(u, 1)], ridxa_v)
        pltpu.sync_copy(rows_hbm.at[pl.ds(u + 1, 1)], ridxb_v)
        pltpu.sync_copy(cols_hbm.at[pl.ds(u, 1)], cidxa_v)
        pltpu.sync_copy(cols_hbm.at[pl.ds(u + 1, 1)], cidxb_v)
        ga = pltpu.async_copy(g_hbm.at[ridxa_v.at[0]], gbufa_v, sga)
        gb = pltpu.async_copy(g_hbm.at[ridxb_v.at[0]], gbufb_v, sgb)
        ga.wait()
        pltpu.sync_copy(gbufa_v, agg_sh.at[cidxa_v.at[0]], add=True)
        gb.wait()
        pltpu.sync_copy(gbufb_v, agg_sh.at[cidxb_v.at[0]], add=True)
        return 0

    lax.fori_loop(0, nu // 2, edge_pair, 0)

    @pl.when(nu % 2 == 1)
    def _tail():
        u = u0 + nu - 1
        pltpu.sync_copy(rows_hbm.at[pl.ds(u, 1)], ridxa_v)
        pltpu.sync_copy(cols_hbm.at[pl.ds(u, 1)], cidxa_v)
        pltpu.async_copy(g_hbm.at[ridxa_v.at[0]], gbufa_v, sga).wait()
        pltpu.sync_copy(gbufa_v, agg_sh.at[cidxa_v.at[0]], add=True)

    plsc.subcore_barrier()

    lr = _row_base(s)
    for off in ROW_OFFS:
        gr = c * HALF + lr + off
        pltpu.sync_copy(agg_sh.at[pl.ds(lr + off, RCH)], agg_v)
        pltpu.sync_copy(g_hbm.at[pl.ds(gr, RCH)], gold_v)
        pltpu.sync_copy(c_hbm.at[pl.ds(gr, RCH)], cvec_v)
        pltpu.sync_copy(a_hbm.at[pl.ds(gr, RCH)], avec_v)

        def rowfn(r, _):
            a_s = avec_v[r, :][0]
            for k in range(D // 16):
                sl = pl.ds(k * 16, 16)
                gnew_v[r, sl] = a_s * (agg_v[r, sl] + gold_v[r, sl]) \
                    + cvec_v[r, sl]
            return 0

        lax.fori_loop(0, RCH, rowfn, 0)
        pltpu.sync_copy(gnew_v, gout_hbm.at[pl.ds(gr, RCH)])


BLK = 1000


def _mlp_body(x_ref, deg_ref, w1_ref, b1_ref, w2_ref, b2_ref,
              g0_ref, a_ref, c_ref, dinv_ref):
    x = x_ref[...]
    h = jnp.dot(x, w1_ref[...].T, preferred_element_type=jnp.float32)
    h = jnp.maximum(h + b1_ref[...], 0.0)
    h = jnp.dot(h, w2_ref[...].T, preferred_element_type=jnp.float32)
    h = h + b2_ref[...]
    deg = deg_ref[...][:, 0:1] + 1.0
    dinv = lax.rsqrt(deg)
    g0 = h * dinv
    g0_ref[...] = g0
    a_ref[...] = jnp.broadcast_to((1.0 - ALPHA) * dinv * dinv, (BLK, 16))
    c_ref[...] = ALPHA * g0
    dinv_ref[...] = dinv


def _mlp_stage(x, deg16, W1, b1, W2, b2):
    grid = (N // BLK,)
    return pl.pallas_call(
        _mlp_body,
        grid=grid,
        in_specs=[
            pl.BlockSpec((BLK, D), lambda i: (i, 0)),
            pl.BlockSpec((BLK, 16), lambda i: (i, 0)),
            pl.BlockSpec((D, D), lambda i: (0, 0)),
            pl.BlockSpec((1, D), lambda i: (0, 0)),
            pl.BlockSpec((D, D), lambda i: (0, 0)),
            pl.BlockSpec((1, D), lambda i: (0, 0)),
        ],
        out_specs=[
            pl.BlockSpec((BLK, D), lambda i: (i, 0)),
            pl.BlockSpec((BLK, 16), lambda i: (i, 0)),
            pl.BlockSpec((BLK, D), lambda i: (i, 0)),
            pl.BlockSpec((BLK, 1), lambda i: (i, 0)),
        ],
        out_shape=[
            jax.ShapeDtypeStruct((N, D), jnp.float32),
            jax.ShapeDtypeStruct((N, 16), jnp.float32),
            jax.ShapeDtypeStruct((N, D), jnp.float32),
            jax.ShapeDtypeStruct((N, 1), jnp.float32),
        ],
    )(x, deg16, W1, b1.reshape(1, D), W2, b2.reshape(1, D))


def _out_body(g_ref, dinv_ref, o_ref):
    h = g_ref[...] / dinv_ref[...]
    m = jnp.max(h, axis=1, keepdims=True)
    ex = jnp.exp(h - m)
    lse = jnp.log(jnp.sum(ex, axis=1, keepdims=True))
    o_ref[...] = h - m - lse


def _out_stage(g, dinv):
    grid = (N // BLK,)
    return pl.pallas_call(
        _out_body,
        grid=grid,
        in_specs=[
            pl.BlockSpec((BLK, D), lambda i: (i, 0)),
            pl.BlockSpec((BLK, 1), lambda i: (i, 0)),
        ],
        out_specs=pl.BlockSpec((BLK, D), lambda i: (i, 0)),
        out_shape=jax.ShapeDtypeStruct((N, D), jnp.float32),
    )(g, dinv)


def _prep_edges(edge_index):
    row = edge_index[0].astype(jnp.int32)
    col = edge_index[1].astype(jnp.int32)
    in0 = col < HALF
    n0 = jnp.sum(in0.astype(jnp.int32))
    pad0 = (-n0) % U
    b0p = n0 + pad0
    pos0 = jnp.cumsum(in0.astype(jnp.int32)) - 1
    pos1 = b0p + jnp.cumsum((~in0).astype(jnp.int32)) - 1
    pos = jnp.where(in0, pos0, pos1)
    rows_p = jnp.zeros((E_PAD,), jnp.int32).at[pos].set(row)
    col_local = col - jnp.where(in0, 0, HALF).astype(jnp.int32)
    cols_p = jnp.full((E_PAD,), GARBAGE, jnp.int32).at[pos].set(col_local)

    t0 = b0p // U
    t1 = EPU - t0
    sar = jnp.arange(NS + 1, dtype=jnp.int32)
    bnd0 = (sar * t0) // NS
    bnd1 = t0 + (sar * t1) // NS
    starts = jnp.concatenate([bnd0[:-1], bnd1[:-1]])
    nums = jnp.concatenate([bnd0[1:] - bnd0[:-1], bnd1[1:] - bnd1[:-1]])
    meta = jnp.zeros((32, 16), jnp.int32)
    meta = meta.at[:, 0].set(starts).at[:, 1].set(nums)
    return (rows_p.reshape(EPU, U), cols_p.reshape(EPU, U), meta)


def kernel(x, edge_index, W1, b1, W2, b2):
    rows2d, cols2d, meta = _prep_edges(edge_index)
    deg16 = _deg_kernel(cols2d, meta)
    g, avec, cvec, dinv = _mlp_stage(x, deg16, W1, b1, W2, b2)
    for _ in range(K):
        g = _prop_kernel(g, rows2d, cols2d, avec, cvec, meta)
    return _out_stage(g, dinv)
